# Initial kernel scaffold; baseline (speedup 1.0000x reference)
import jax
import jax.numpy as jnp
from jax.experimental import pallas as pl


def kernel(x, edge_index, batch, fc0_W, fc0_b, Wr1, Wn1, b1, Wr2, Wn2, b2, Wr3, Wn3, b3, fc1_W, fc1_b, fc2_W, fc2_b):
    raise NotImplementedError("write your pallas kernel here")


# speedup vs baseline: 7.5615x; 7.5615x over previous
import functools

import jax
import jax.numpy as jnp
from jax import lax
from jax.experimental import pallas as pl
from jax.experimental.pallas import tpu as pltpu
from jax.experimental.pallas import tpu_sc as plsc

N_NODES = 50000
NPAD = 50048
EDGES = 800000
EP = 819200
LANES = 16
NCORES = 2
NSUB = 16
CHUNK = 128
NJ = 4
IDXG = 8


def _segsum_sc(table, srcs, dsts, d_eff):
    TR = srcs.shape[0]
    PT = TR // (NCORES * NSUB)
    OUTER = PT // IDXG
    assert PT % IDXG == 0 and TR % (NCORES * NSUB) == 0
    stripe = NPAD // NSUB
    ZR = 136
    nzero = stripe // ZR
    assert stripe % ZR == 0
    mesh = plsc.VectorSubcoreMesh(core_axis_name="c", subcore_axis_name="s")

    @functools.partial(
        pl.kernel,
        out_type=jax.ShapeDtypeStruct((NCORES, NPAD, d_eff), jnp.float32),
        mesh=mesh,
        scratch_types=[pltpu.VMEM_SHARED((NPAD, d_eff), jnp.float32),
                       pltpu.VMEM((IDXG, CHUNK), jnp.int32),
                       pltpu.VMEM((IDXG, CHUNK), jnp.int32)]
                      + [pltpu.VMEM((CHUNK, d_eff), jnp.float32)
                         for _ in range(NJ)]
                      + [pltpu.VMEM((ZR, d_eff), jnp.float32),
                         pltpu.SemaphoreType.DMA],
        compiler_params=pltpu.CompilerParams(use_tc_tiling_on_sc=False),
    )
    def seg_kernel(table_h, srcs_h, dsts_h, out_h, acc_s, srcv, dstv, *rest):
        rows = rest[:NJ]
        zv = rest[NJ]
        gsem = rest[NJ + 1]
        c = lax.axis_index("c")
        s = lax.axis_index("s")

        zero16 = jnp.zeros((LANES,), jnp.float32)

        @pl.loop(0, ZR)
        def _(r):
            for kk in range(d_eff // LANES):
                zv[r, pl.ds(kk * LANES, LANES)] = zero16

        r0 = s * stripe

        @pl.loop(0, nzero)
        def _(i):
            pltpu.sync_copy(zv, acc_s.at[pl.ds(r0 + i * ZR, ZR)])

        plsc.subcore_barrier()

        base = c * (TR // 2) + s * PT

        @pl.loop(0, OUTER)
        def _(o):
            row0 = base + o * IDXG
            pltpu.sync_copy(srcs_h.at[pl.ds(row0, IDXG)], srcv)
            pltpu.sync_copy(dsts_h.at[pl.ds(row0, IDXG)], dstv)
            for g in range(IDXG // NJ):
                handles = [
                    pltpu.async_copy(table_h.at[srcv.at[g * NJ + j]],
                                     rows[j], gsem)
                    for j in range(NJ)
                ]
                for j in range(NJ):
                    handles[j].wait()
                    pltpu.sync_copy(rows[j],
                                    acc_s.at[dstv.at[g * NJ + j]], add=True)

        plsc.subcore_barrier()
        pltpu.sync_copy(acc_s.at[pl.ds(r0, stripe)],
                        out_h.at[c, pl.ds(r0, stripe)])

    return seg_kernel(table, srcs, dsts)


def _elu(x):
    return jnp.where(x > 0, x, jnp.expm1(x))


def kernel(x, edge_index, batch, fc0_W, fc0_b, Wr1, Wn1, b1, Wr2, Wn2, b2,
           Wr3, Wn3, b3, fc1_W, fc1_b, fc2_W, fc2_b):
    pad = EP - EDGES
    src = edge_index[0]
    dst = edge_index[1]
    src_p = jnp.concatenate(
        [src, jnp.zeros((pad,), jnp.int32)]).reshape(EP // CHUNK, CHUNK)
    dst_p = jnp.concatenate(
        [dst, jnp.full((pad,), N_NODES, jnp.int32)]).reshape(EP // CHUNK, CHUNK)
    src_f = jnp.concatenate([src_p, src_p + N_NODES], axis=0)
    dst_f = jnp.concatenate([dst_p, dst_p], axis=0)

    h = _elu(x @ fc0_W + fc0_b)

    p = _segsum_sc(h, src_p, dst_p, 16)
    agg = (p[0] + p[1])[:N_NODES]
    h = _elu(h @ Wr1 + agg @ Wn1 + b1)

    p = _segsum_sc(h, src_p, dst_p, 32)
    agg = (p[0] + p[1])[:N_NODES]
    h = _elu(h @ Wr2 + agg @ Wn2 + b2)

    table = jnp.concatenate([h[:, :32], h[:, 32:]], axis=0)
    p = _segsum_sc(table, src_f, dst_f, 32)
    agg = jnp.concatenate([p[0], p[1]], axis=1)[:N_NODES]
    h = _elu(h @ Wr3 + agg @ Wn3 + b3)

    h = _elu(h @ fc1_W + fc1_b)
    h = h @ fc2_W + fc2_b
    return jax.nn.log_softmax(h, axis=1)


# speedup vs baseline: 11.2832x; 1.4922x over previous
import functools

import jax
import jax.numpy as jnp
from jax import lax
from jax.experimental import pallas as pl
from jax.experimental.pallas import tpu as pltpu
from jax.experimental.pallas import tpu_sc as plsc

N_NODES = 50000
NPAD = 50048
EDGES = 800000
EP = 819200
LANES = 16
NCORES = 2
NSUB = 16
CHUNK = 128
NJ = 4
IDXG = 8


def _segsum_sc(table, srcs, dsts, d_eff):
    TR = srcs.shape[0]
    PT = TR // (NCORES * NSUB)
    OUTER = PT // IDXG
    assert PT % IDXG == 0 and TR % (NCORES * NSUB) == 0
    stripe = NPAD // NSUB
    ZR = 136
    nzero = stripe // ZR
    assert stripe % ZR == 0
    mesh = plsc.VectorSubcoreMesh(core_axis_name="c", subcore_axis_name="s")

    @functools.partial(
        pl.kernel,
        out_type=jax.ShapeDtypeStruct((NCORES, NPAD, d_eff), jnp.float32),
        mesh=mesh,
        scratch_types=[pltpu.VMEM_SHARED((NPAD, d_eff), jnp.float32),
                       pltpu.VMEM((IDXG, CHUNK), jnp.int32),
                       pltpu.VMEM((IDXG, CHUNK), jnp.int32)]
                      + [pltpu.VMEM((CHUNK, d_eff), jnp.float32)
                         for _ in range(NJ)]
                      + [pltpu.VMEM((ZR, d_eff), jnp.float32),
                         pltpu.SemaphoreType.DMA],
        compiler_params=pltpu.CompilerParams(use_tc_tiling_on_sc=False),
    )
    def seg_kernel(table_h, srcs_h, dsts_h, out_h, acc_s, srcv, dstv, *rest):
        rows = rest[:NJ]
        zv = rest[NJ]
        gsem = rest[NJ + 1]
        c = lax.axis_index("c")
        s = lax.axis_index("s")

        zero16 = jnp.zeros((LANES,), jnp.float32)

        @pl.loop(0, ZR)
        def _(r):
            for kk in range(d_eff // LANES):
                zv[r, pl.ds(kk * LANES, LANES)] = zero16

        r0 = s * stripe

        @pl.loop(0, nzero)
        def _(i):
            pltpu.sync_copy(zv, acc_s.at[pl.ds(r0 + i * ZR, ZR)])

        plsc.subcore_barrier()

        base = c * (TR // 2) + s * PT

        @pl.loop(0, OUTER)
        def _(o):
            row0 = base + o * IDXG
            pltpu.sync_copy(srcs_h.at[pl.ds(row0, IDXG)], srcv)
            pltpu.sync_copy(dsts_h.at[pl.ds(row0, IDXG)], dstv)
            for g in range(IDXG // NJ):
                handles = [
                    pltpu.async_copy(table_h.at[srcv.at[g * NJ + j]],
                                     rows[j], gsem)
                    for j in range(NJ)
                ]
                for j in range(NJ):
                    handles[j].wait()
                    pltpu.sync_copy(rows[j],
                                    acc_s.at[dstv.at[g * NJ + j]], add=True)

        plsc.subcore_barrier()
        pltpu.sync_copy(acc_s.at[pl.ds(r0, stripe)],
                        out_h.at[c, pl.ds(r0, stripe)])

    return seg_kernel(table, srcs, dsts)


def _elu(x):
    return jnp.where(x > 0, x, jnp.expm1(x))


def kernel(x, edge_index, batch, fc0_W, fc0_b, Wr1, Wn1, b1, Wr2, Wn2, b2,
           Wr3, Wn3, b3, fc1_W, fc1_b, fc2_W, fc2_b):
    pad = EP - EDGES
    src = edge_index[0]
    dst = edge_index[1]
    pad_iota = jnp.arange(pad, dtype=jnp.int32)
    src_p = jnp.concatenate(
        [src, pad_iota % N_NODES]).reshape(EP // CHUNK, CHUNK)
    dst_p = jnp.concatenate(
        [dst, N_NODES + pad_iota % (NPAD - N_NODES)]).reshape(
            EP // CHUNK, CHUNK)
    src_f = jnp.concatenate([src_p, src_p + N_NODES], axis=0)
    dst_f = jnp.concatenate([dst_p, dst_p], axis=0)

    h = _elu(x @ fc0_W + fc0_b)

    p = _segsum_sc(h, src_p, dst_p, 16)
    agg = (p[0] + p[1])[:N_NODES]
    h = _elu(h @ Wr1 + agg @ Wn1 + b1)

    p = _segsum_sc(h, src_p, dst_p, 32)
    agg = (p[0] + p[1])[:N_NODES]
    h = _elu(h @ Wr2 + agg @ Wn2 + b2)

    table = jnp.concatenate([h[:, :32], h[:, 32:]], axis=0)
    p = _segsum_sc(table, src_f, dst_f, 32)
    agg = jnp.concatenate([p[0], p[1]], axis=1)[:N_NODES]
    h = _elu(h @ Wr3 + agg @ Wn3 + b3)

    h = _elu(h @ fc1_W + fc1_b)
    h = h @ fc2_W + fc2_b
    return jax.nn.log_softmax(h, axis=1)


# speedup vs baseline: 12.6492x; 1.1211x over previous
import functools

import jax
import jax.numpy as jnp
from jax import lax
from jax.experimental import pallas as pl
from jax.experimental.pallas import tpu as pltpu
from jax.experimental.pallas import tpu_sc as plsc

N_NODES = 50000
NPAD = 50048
EDGES = 800000
EP = 811008
LANES = 16
NCORES = 2
NSUB = 16
CHUNK = 96
NJ = 8


def _segsum_sc(table, sd, d_eff):
    TR = sd.shape[0] // 2
    PT = TR // (NCORES * NSUB)
    NB = PT // NJ
    assert PT % NJ == 0
    PAIRS, TAIL = NB // 2, NB % 2
    stripe = NPAD // NSUB
    ZFULL, ZTAIL = stripe // CHUNK, stripe % CHUNK
    assert ZTAIL % 8 == 0
    mesh = plsc.VectorSubcoreMesh(core_axis_name="c", subcore_axis_name="s")

    @functools.partial(
        pl.kernel,
        out_type=jax.ShapeDtypeStruct((NCORES, NPAD, d_eff), jnp.float32),
        mesh=mesh,
        scratch_types=[pltpu.VMEM_SHARED((NPAD, d_eff), jnp.float32),
                       pltpu.VMEM((2 * NJ, CHUNK), jnp.int32),
                       pltpu.VMEM((2 * NJ, CHUNK), jnp.int32)]
                      + [pltpu.VMEM((CHUNK, d_eff), jnp.float32)
                         for _ in range(NJ)]
                      + [pltpu.SemaphoreType.DMA,
                         pltpu.SemaphoreType.DMA,
                         pltpu.SemaphoreType.DMA],
        compiler_params=pltpu.CompilerParams(use_tc_tiling_on_sc=False),
    )
    def seg_kernel(table_h, sd_h, out_h, acc_s, idx0, idx1, *rest):
        rows = rest[:NJ]
        isem, gsem, ssem = rest[NJ], rest[NJ + 1], rest[NJ + 2]
        c = lax.axis_index("c")
        s = lax.axis_index("s")

        zero16 = jnp.zeros((LANES,), jnp.float32)

        @pl.loop(0, CHUNK)
        def _(r):
            for k in range(d_eff // LANES):
                rows[0][r, pl.ds(k * LANES, LANES)] = zero16
                rows[1][r, pl.ds(k * LANES, LANES)] = zero16

        r0 = s * stripe
        zh = []
        for i in range(ZFULL):
            zh.append(pltpu.async_copy(
                rows[i % 2], acc_s.at[pl.ds(r0 + i * CHUNK, CHUNK)], gsem))
        zh.append(pltpu.async_copy(
            rows[0].at[pl.ds(0, ZTAIL)],
            acc_s.at[pl.ds(r0 + ZFULL * CHUNK, ZTAIL)], ssem))
        for h in zh:
            h.wait()

        plsc.subcore_barrier()

        base = c * (TR // 2) + s * PT
        sdbase = 2 * base

        def load_block(buf, b):
            return pltpu.async_copy(
                sd_h.at[pl.ds(sdbase + 2 * NJ * b, 2 * NJ)], buf, isem)

        def process(buf, prefetch):
            gh = [pltpu.async_copy(table_h.at[buf.at[2 * k]], rows[k], gsem)
                  for k in range(NJ)]
            ph = prefetch() if prefetch is not None else None
            sh = []
            for k in range(NJ):
                gh[k].wait()
                sh.append(pltpu.async_copy(
                    rows[k], acc_s.at[buf.at[2 * k + 1]], ssem, add=True))
            for h in sh:
                h.wait()
            if ph is not None:
                ph.wait()

        load_block(idx0, 0).wait()

        @pl.loop(0, PAIRS)
        def _(o2):
            b0 = 2 * o2
            process(idx0, lambda: load_block(idx1, b0 + 1))
            nxt = lax.min(b0 + 2, NB - 1)
            process(idx1, lambda: load_block(idx0, nxt))

        if TAIL:
            process(idx0, None)

        plsc.subcore_barrier()
        pltpu.sync_copy(acc_s.at[pl.ds(r0, stripe)],
                        out_h.at[c, pl.ds(r0, stripe)])

    return seg_kernel(table, sd)


def _elu(x):
    return jnp.where(x > 0, x, jnp.expm1(x))


def kernel(x, edge_index, batch, fc0_W, fc0_b, Wr1, Wn1, b1, Wr2, Wn2, b2,
           Wr3, Wn3, b3, fc1_W, fc1_b, fc2_W, fc2_b):
    pad = EP - EDGES
    src = edge_index[0]
    dst = edge_index[1]
    pad_iota = jnp.arange(pad, dtype=jnp.int32)
    src_p = jnp.concatenate([src, pad_iota % N_NODES])
    dst_p = jnp.concatenate([dst, N_NODES + pad_iota % (NPAD - N_NODES)])

    def chunked(sv, dv):
        return jnp.stack([sv.reshape(EP // CHUNK, CHUNK),
                          dv.reshape(EP // CHUNK, CHUNK)],
                         axis=1).reshape(2 * EP // CHUNK, CHUNK)

    sd_e = chunked(src_p, dst_p)
    sd_f = jnp.concatenate([sd_e, chunked(src_p + N_NODES, dst_p)], axis=0)

    h = _elu(x @ fc0_W + fc0_b)

    p = _segsum_sc(h, sd_e, 16)
    agg = (p[0] + p[1])[:N_NODES]
    h = _elu(h @ Wr1 + agg @ Wn1 + b1)

    p = _segsum_sc(h, sd_e, 32)
    agg = (p[0] + p[1])[:N_NODES]
    h = _elu(h @ Wr2 + agg @ Wn2 + b2)

    table = jnp.concatenate([h[:, :32], h[:, 32:]], axis=0)
    p = _segsum_sc(table, sd_f, 32)
    agg = jnp.concatenate([p[0], p[1]], axis=1)[:N_NODES]
    h = _elu(h @ Wr3 + agg @ Wn3 + b3)

    h = _elu(h @ fc1_W + fc1_b)
    h = h @ fc2_W + fc2_b
    return jax.nn.log_softmax(h, axis=1)


# speedup vs baseline: 13.2726x; 1.0493x over previous
import functools

import jax
import jax.numpy as jnp
from jax import lax
from jax.experimental import pallas as pl
from jax.experimental.pallas import tpu as pltpu
from jax.experimental.pallas import tpu_sc as plsc

N_NODES = 50000
NPAD = 50048
EDGES = 800000
EP = 811008
LANES = 16
NCORES = 2
NSUB = 16
CHUNK = 96
NJ = 8


def _segsum_sc(table_a, table_b, sd, d_eff, full_range):
    TR = sd.shape[0] // 2
    PT = TR // NSUB if full_range else TR // (NCORES * NSUB)
    NB = PT // NJ
    assert PT % NJ == 0
    PAIRS, TAIL = NB // 2, NB % 2
    stripe = NPAD // NSUB
    ZFULL, ZTAIL = stripe // CHUNK, stripe % CHUNK
    assert ZTAIL % 8 == 0
    mesh = plsc.VectorSubcoreMesh(core_axis_name="c", subcore_axis_name="s")

    @functools.partial(
        pl.kernel,
        out_type=jax.ShapeDtypeStruct((NCORES, NPAD, d_eff), jnp.float32),
        mesh=mesh,
        scratch_types=[pltpu.VMEM_SHARED((NPAD, d_eff), jnp.float32),
                       pltpu.VMEM((2 * NJ, CHUNK), jnp.int32),
                       pltpu.VMEM((2 * NJ, CHUNK), jnp.int32)]
                      + [pltpu.VMEM((CHUNK, d_eff), jnp.float32)
                         for _ in range(NJ)]
                      + [pltpu.SemaphoreType.DMA,
                         pltpu.SemaphoreType.DMA,
                         pltpu.SemaphoreType.DMA],
        compiler_params=pltpu.CompilerParams(use_tc_tiling_on_sc=False),
    )
    def seg_kernel(ta_h, tb_h, sd_h, out_h, acc_s, idx0, idx1, *rest):
        rows = rest[:NJ]
        isem, gsem, ssem = rest[NJ], rest[NJ + 1], rest[NJ + 2]
        c = lax.axis_index("c")
        s = lax.axis_index("s")

        zero16 = jnp.zeros((LANES,), jnp.float32)

        @pl.loop(0, CHUNK)
        def _(r):
            for k in range(d_eff // LANES):
                rows[0][r, pl.ds(k * LANES, LANES)] = zero16
                rows[1][r, pl.ds(k * LANES, LANES)] = zero16

        r0 = s * stripe
        zh = []
        for i in range(ZFULL):
            zh.append(pltpu.async_copy(
                rows[i % 2], acc_s.at[pl.ds(r0 + i * CHUNK, CHUNK)], gsem))
        zh.append(pltpu.async_copy(
            rows[0].at[pl.ds(0, ZTAIL)],
            acc_s.at[pl.ds(r0 + ZFULL * CHUNK, ZTAIL)], ssem))
        for h in zh:
            h.wait()

        plsc.subcore_barrier()

        if full_range:
            base = s * PT
        else:
            base = c * (TR // 2) + s * PT
        sdbase = 2 * base

        def load_block(buf, b):
            return pltpu.async_copy(
                sd_h.at[pl.ds(sdbase + 2 * NJ * b, 2 * NJ)], buf, isem)

        def main_loop(table_h):
            def process(buf, prefetch):
                gh = [pltpu.async_copy(table_h.at[buf.at[2 * k]], rows[k],
                                       gsem)
                      for k in range(NJ)]
                ph = prefetch() if prefetch is not None else None
                sh = []
                for k in range(NJ):
                    gh[k].wait()
                    sh.append(pltpu.async_copy(
                        rows[k], acc_s.at[buf.at[2 * k + 1]], ssem, add=True))
                for h in sh:
                    h.wait()
                if ph is not None:
                    ph.wait()

            load_block(idx0, 0).wait()

            @pl.loop(0, PAIRS)
            def _(o2):
                b0 = 2 * o2
                process(idx0, lambda: load_block(idx1, b0 + 1))
                nxt = lax.min(b0 + 2, NB - 1)
                process(idx1, lambda: load_block(idx0, nxt))

            if TAIL:
                process(idx0, None)

        if table_a is table_b:
            main_loop(ta_h)
        else:
            @pl.when(c == 0)
            def _():
                main_loop(ta_h)

            @pl.when(c == 1)
            def _():
                main_loop(tb_h)

        plsc.subcore_barrier()
        pltpu.sync_copy(acc_s.at[pl.ds(r0, stripe)],
                        out_h.at[c, pl.ds(r0, stripe)])

    return seg_kernel(table_a, table_b, sd)


def _elu(x):
    return jnp.where(x > 0, x, jnp.expm1(x))


def kernel(x, edge_index, batch, fc0_W, fc0_b, Wr1, Wn1, b1, Wr2, Wn2, b2,
           Wr3, Wn3, b3, fc1_W, fc1_b, fc2_W, fc2_b):
    pad = EP - EDGES
    src = edge_index[0]
    dst = edge_index[1]
    pad_iota = jnp.arange(pad, dtype=jnp.int32)
    src_p = jnp.concatenate([src, pad_iota % N_NODES])
    dst_p = jnp.concatenate([dst, N_NODES + pad_iota % (NPAD - N_NODES)])

    sd_e = jnp.stack([src_p.reshape(EP // CHUNK, CHUNK),
                      dst_p.reshape(EP // CHUNK, CHUNK)],
                     axis=1).reshape(2 * EP // CHUNK, CHUNK)

    h = _elu(x @ fc0_W + fc0_b)

    p = _segsum_sc(h, h, sd_e, 16, False)
    agg = (p[0] + p[1])[:N_NODES]
    h = _elu(h @ Wr1 + agg @ Wn1 + b1)

    p = _segsum_sc(h, h, sd_e, 32, False)
    agg = (p[0] + p[1])[:N_NODES]
    ha = _elu(h @ Wr2[:, :32] + agg @ Wn2[:, :32] + b2[:32])
    hb = _elu(h @ Wr2[:, 32:] + agg @ Wn2[:, 32:] + b2[32:])

    p = _segsum_sc(ha, hb, sd_e, 32, True)
    h = _elu(ha @ Wr3[:32] + hb @ Wr3[32:]
             + p[0][:N_NODES] @ Wn3[:32] + p[1][:N_NODES] @ Wn3[32:]
             + b3)

    h = _elu(h @ fc1_W + fc1_b)
    h = h @ fc2_W + fc2_b
    return jax.nn.log_softmax(h, axis=1)
